# Initial kernel scaffold; baseline (speedup 1.0000x reference)
import jax
import jax.numpy as jnp
from jax.experimental import pallas as pl


def kernel(a_ij, coords, edge_index, W1, b1, W2, b2, Wh):
    raise NotImplementedError("write your pallas kernel here")


# speedup vs baseline: 9.1295x; 9.1295x over previous
import functools

import jax
import jax.numpy as jnp
from jax import lax
from jax.experimental import pallas as pl
from jax.experimental.pallas import tpu as pltpu
from jax.experimental.pallas import tpu_sc as plsc


def _att_body(a_ref, w1_ref, b1_ref, w2_ref, b2_ref, wh_ref, o_ref):
    h = jnp.dot(a_ref[...], w1_ref[...], preferred_element_type=jnp.float32)
    h = h + b1_ref[...]
    h = jnp.where(h >= 0.0, h, 0.01 * h)
    v = jnp.dot(w2_ref[...], wh_ref[...], preferred_element_type=jnp.float32)
    c = jnp.dot(b2_ref[...], wh_ref[...], preferred_element_type=jnp.float32)
    att = jnp.dot(h, v, preferred_element_type=jnp.float32) + c
    o_ref[...] = att.reshape(1, 1, att.shape[0])


def _compute_att(a_ij, W1, b1, W2, b2, Wh, block_e):
    e = a_ij.shape[0]
    nb = e // block_e
    out = pl.pallas_call(
        _att_body,
        grid=(nb,),
        in_specs=[
            pl.BlockSpec((block_e, a_ij.shape[1]), lambda g: (g, 0)),
            pl.BlockSpec(W1.shape, lambda g: (0, 0)),
            pl.BlockSpec((1, b1.shape[0]), lambda g: (0, 0)),
            pl.BlockSpec(W2.shape, lambda g: (0, 0)),
            pl.BlockSpec((1, b2.shape[0]), lambda g: (0, 0)),
            pl.BlockSpec(Wh.shape, lambda g: (0, 0)),
        ],
        out_specs=pl.BlockSpec((1, 1, block_e), lambda g: (g, 0, 0)),
        out_shape=jax.ShapeDtypeStruct((nb, 1, block_e), jnp.float32),
    )(a_ij, W1, b1.reshape(1, -1), W2, b2.reshape(1, -1), Wh)
    return out.reshape(e)


_LANES = 16
_MAGIC = 0x5F3759DF


def _rsqrt16(x):
    i = plsc.bitcast(x, jnp.int32)
    i = _MAGIC - lax.shift_right_logical(i, 1)
    y = plsc.bitcast(i, jnp.float32)
    hx = 0.5 * x
    y = y * (1.5 - hx * y * y)
    y = y * (1.5 - hx * y * y)
    y = y * (1.5 - hx * y * y)
    return y


def _make_sc_edge(n, e, n_workers):
    ew = e // n_workers
    cw = 3 * n
    mesh = plsc.VectorSubcoreMesh(core_axis_name="c", subcore_axis_name="s")

    @functools.partial(
        pl.kernel,
        mesh=mesh,
        compiler_params=pltpu.CompilerParams(needs_layout_passes=False),
        out_type=jax.ShapeDtypeStruct((n_workers, cw), jnp.float32),
        scratch_types=[
            pltpu.VMEM((cw,), jnp.float32),
            pltpu.VMEM((cw,), jnp.float32),
            pltpu.VMEM((ew,), jnp.int32),
            pltpu.VMEM((ew,), jnp.int32),
            pltpu.VMEM((ew,), jnp.float32),
        ],
    )
    def sc_edge(coords_hbm, i_hbm, j_hbm, att_hbm, out_hbm,
                coords_v, acc_v, i_v, j_v, att_v):
        cid = lax.axis_index("c")
        sid = lax.axis_index("s")
        wid = sid * 2 + cid
        base = pl.multiple_of(wid * ew, 8)

        pltpu.sync_copy(coords_hbm, coords_v)
        pltpu.sync_copy(i_hbm.at[pl.ds(base, ew)], i_v)
        pltpu.sync_copy(j_hbm.at[pl.ds(base, ew)], j_v)
        pltpu.sync_copy(att_hbm.at[pl.ds(base, ew)], att_v)

        zeros = jnp.zeros((_LANES,), jnp.float32)

        def zero_body(t, _):
            acc_v[pl.ds(t * _LANES, _LANES)] = zeros
            return 0

        lax.fori_loop(0, cw // _LANES, zero_body, 0)

        def edge_body(t, _):
            off = t * _LANES
            iv = i_v[pl.ds(off, _LANES)]
            jv = j_v[pl.ds(off, _LANES)]
            av = att_v[pl.ds(off, _LANES)]
            bi = iv * 3
            bj = jv * 3
            xi = plsc.load_gather(coords_v, [bi])
            yi = plsc.load_gather(coords_v, [bi + 1])
            zi = plsc.load_gather(coords_v, [bi + 2])
            xj = plsc.load_gather(coords_v, [bj])
            yj = plsc.load_gather(coords_v, [bj + 1])
            zj = plsc.load_gather(coords_v, [bj + 2])
            dx = xi - xj
            dy = yi - yj
            dz = zi - zj
            s2 = dx * dx + dy * dy + dz * dz
            s2 = jnp.maximum(s2, 1e-30)
            norm = s2 * _rsqrt16(s2)
            f = av / (norm + 1e-6)
            plsc.addupdate_scatter(acc_v, [bi], dx * f)
            plsc.addupdate_scatter(acc_v, [bi + 1], dy * f)
            plsc.addupdate_scatter(acc_v, [bi + 2], dz * f)
            return 0

        lax.fori_loop(0, ew // _LANES, edge_body, 0)

        pltpu.sync_copy(acc_v, out_hbm.at[wid])

    return sc_edge


def _reduce_body(p_ref, c_ref, o_ref):
    o_ref[...] = c_ref[...] + jnp.sum(p_ref[...], axis=0, keepdims=True)


def _reduce_partials(partials, coords_flat):
    nw, cw = partials.shape
    out = pl.pallas_call(
        _reduce_body,
        in_specs=[
            pl.BlockSpec((nw, cw), lambda: (0, 0)),
            pl.BlockSpec((1, cw), lambda: (0, 0)),
        ],
        out_specs=pl.BlockSpec((1, cw), lambda: (0, 0)),
        out_shape=jax.ShapeDtypeStruct((1, cw), jnp.float32),
    )(partials, coords_flat.reshape(1, cw))
    return out.reshape(cw)


def kernel(a_ij, coords, edge_index, W1, b1, W2, b2, Wh):
    e = a_ij.shape[0]
    n = coords.shape[0]
    att = _compute_att(a_ij, W1, b1, W2, b2, Wh, block_e=2560)
    coords_flat = coords.reshape(-1)
    sc_edge = _make_sc_edge(n, e, 32)
    partials = sc_edge(coords_flat, edge_index[0], edge_index[1], att)
    out_flat = _reduce_partials(partials, coords_flat)
    return out_flat.reshape(n, 3)


# speedup vs baseline: 11.8007x; 1.2926x over previous
import functools

import jax
import jax.numpy as jnp
from jax import lax
from jax.experimental import pallas as pl
from jax.experimental.pallas import tpu as pltpu
from jax.experimental.pallas import tpu_sc as plsc


def _att_body(a_ref, w1_ref, b1_ref, w2_ref, b2_ref, wh_ref, o_ref):
    h = jnp.dot(a_ref[...], w1_ref[...], preferred_element_type=jnp.float32)
    h = h + b1_ref[...]
    h = jnp.where(h >= 0.0, h, 0.01 * h)
    v = jnp.dot(w2_ref[...], wh_ref[...], preferred_element_type=jnp.float32)
    c = jnp.dot(b2_ref[...], wh_ref[...], preferred_element_type=jnp.float32)
    ht = h.T
    att = jnp.dot(v.T, ht, preferred_element_type=jnp.float32) + c
    o_ref[...] = att.reshape(1, 1, att.shape[1])


def _compute_att(a_ij, W1, b1, W2, b2, Wh, block_e):
    e = a_ij.shape[0]
    nb = e // block_e
    out = pl.pallas_call(
        _att_body,
        grid=(nb,),
        in_specs=[
            pl.BlockSpec((block_e, a_ij.shape[1]), lambda g: (g, 0)),
            pl.BlockSpec(W1.shape, lambda g: (0, 0)),
            pl.BlockSpec((1, b1.shape[0]), lambda g: (0, 0)),
            pl.BlockSpec(W2.shape, lambda g: (0, 0)),
            pl.BlockSpec((1, b2.shape[0]), lambda g: (0, 0)),
            pl.BlockSpec(Wh.shape, lambda g: (0, 0)),
        ],
        out_specs=pl.BlockSpec((1, 1, block_e), lambda g: (g, 0, 0)),
        out_shape=jax.ShapeDtypeStruct((nb, 1, block_e), jnp.float32),
    )(a_ij, W1, b1.reshape(1, -1), W2, b2.reshape(1, -1), Wh)
    return out.reshape(e)


_LANES = 16
_MAGIC = 0x5F3759DF


def _rsqrt16(x):
    i = plsc.bitcast(x, jnp.int32)
    i = _MAGIC - lax.shift_right_logical(i, 1)
    y = plsc.bitcast(i, jnp.float32)
    hx = 0.5 * x
    y = y * (1.5 - hx * y * y)
    y = y * (1.5 - hx * y * y)
    y = y * (1.5 - hx * y * y)
    return y


def _make_sc_edge(n, e, n_workers):
    ew = e // n_workers
    cw = 3 * n
    mesh = plsc.VectorSubcoreMesh(core_axis_name="c", subcore_axis_name="s")

    @functools.partial(
        pl.kernel,
        mesh=mesh,
        compiler_params=pltpu.CompilerParams(needs_layout_passes=False),
        out_type=jax.ShapeDtypeStruct((n_workers, cw), jnp.float32),
        scratch_types=[
            pltpu.VMEM((cw,), jnp.float32),
            pltpu.VMEM((cw,), jnp.float32),
            pltpu.VMEM((ew,), jnp.int32),
            pltpu.VMEM((ew,), jnp.int32),
            pltpu.VMEM((ew,), jnp.float32),
        ],
    )
    def sc_edge(coords_hbm, i_hbm, j_hbm, att_hbm, out_hbm,
                coords_v, acc_v, i_v, j_v, att_v):
        cid = lax.axis_index("c")
        sid = lax.axis_index("s")
        wid = sid * 2 + cid
        base = pl.multiple_of(wid * ew, 8)

        pltpu.sync_copy(coords_hbm, coords_v)
        pltpu.sync_copy(i_hbm.at[pl.ds(base, ew)], i_v)
        pltpu.sync_copy(j_hbm.at[pl.ds(base, ew)], j_v)
        pltpu.sync_copy(att_hbm.at[pl.ds(base, ew)], att_v)

        zeros = jnp.zeros((_LANES,), jnp.float32)

        def zero_body(t, _):
            acc_v[pl.ds(t * _LANES, _LANES)] = zeros
            return 0

        lax.fori_loop(0, cw // _LANES, zero_body, 0)

        def edge_body(t, _):
            off = t * _LANES
            iv = i_v[pl.ds(off, _LANES)]
            jv = j_v[pl.ds(off, _LANES)]
            av = att_v[pl.ds(off, _LANES)]
            bi = iv * 3
            bj = jv * 3
            xi = plsc.load_gather(coords_v, [bi])
            yi = plsc.load_gather(coords_v, [bi + 1])
            zi = plsc.load_gather(coords_v, [bi + 2])
            xj = plsc.load_gather(coords_v, [bj])
            yj = plsc.load_gather(coords_v, [bj + 1])
            zj = plsc.load_gather(coords_v, [bj + 2])
            dx = xi - xj
            dy = yi - yj
            dz = zi - zj
            s2 = dx * dx + dy * dy + dz * dz
            s2 = jnp.maximum(s2, 1e-30)
            norm = s2 * _rsqrt16(s2)
            f = av / (norm + 1e-6)
            plsc.addupdate_scatter(acc_v, [bi], dx * f)
            plsc.addupdate_scatter(acc_v, [bi + 1], dy * f)
            plsc.addupdate_scatter(acc_v, [bi + 2], dz * f)
            return 0

        lax.fori_loop(0, ew // _LANES, edge_body, 0)

        pltpu.sync_copy(acc_v, out_hbm.at[wid])

    return sc_edge


def _reduce_body(p_ref, c_ref, o_ref):
    o_ref[...] = c_ref[...] + jnp.sum(p_ref[...], axis=0, keepdims=True)


def _reduce_partials(partials, coords_flat):
    nw, cw = partials.shape
    out = pl.pallas_call(
        _reduce_body,
        in_specs=[
            pl.BlockSpec((nw, cw), lambda: (0, 0)),
            pl.BlockSpec((1, cw), lambda: (0, 0)),
        ],
        out_specs=pl.BlockSpec((1, cw), lambda: (0, 0)),
        out_shape=jax.ShapeDtypeStruct((1, cw), jnp.float32),
    )(partials, coords_flat.reshape(1, cw))
    return out.reshape(cw)


def kernel(a_ij, coords, edge_index, W1, b1, W2, b2, Wh):
    e = a_ij.shape[0]
    n = coords.shape[0]
    att = _compute_att(a_ij, W1, b1, W2, b2, Wh, block_e=2560)
    coords_flat = coords.reshape(-1)
    sc_edge = _make_sc_edge(n, e, 32)
    partials = sc_edge(coords_flat, edge_index[0], edge_index[1], att)
    out_flat = _reduce_partials(partials, coords_flat)
    return out_flat.reshape(n, 3)


# speedup vs baseline: 13.0446x; 1.1054x over previous
import functools

import jax
import jax.numpy as jnp
from jax import lax
from jax.experimental import pallas as pl
from jax.experimental.pallas import tpu as pltpu
from jax.experimental.pallas import tpu_sc as plsc


def _att_body(a_ref, w1_ref, b1_ref, w2_ref, b2_ref, wh_ref, o_ref):
    h = jnp.dot(a_ref[...], w1_ref[...], preferred_element_type=jnp.float32)
    h = h + b1_ref[...]
    h = jnp.where(h >= 0.0, h, 0.01 * h)
    v = jnp.dot(w2_ref[...], wh_ref[...], preferred_element_type=jnp.float32)
    c = jnp.dot(b2_ref[...], wh_ref[...], preferred_element_type=jnp.float32)
    ht = h.T
    att = jnp.dot(v.T, ht, preferred_element_type=jnp.float32) + c
    o_ref[...] = att.reshape(1, 1, att.shape[1])


def _compute_att(a_ij, W1, b1, W2, b2, Wh, block_e):
    e = a_ij.shape[0]
    nb = e // block_e
    out = pl.pallas_call(
        _att_body,
        grid=(nb,),
        in_specs=[
            pl.BlockSpec((block_e, a_ij.shape[1]), lambda g: (g, 0)),
            pl.BlockSpec(W1.shape, lambda g: (0, 0)),
            pl.BlockSpec((1, b1.shape[0]), lambda g: (0, 0)),
            pl.BlockSpec(W2.shape, lambda g: (0, 0)),
            pl.BlockSpec((1, b2.shape[0]), lambda g: (0, 0)),
            pl.BlockSpec(Wh.shape, lambda g: (0, 0)),
        ],
        out_specs=pl.BlockSpec((1, 1, block_e), lambda g: (g, 0, 0)),
        out_shape=jax.ShapeDtypeStruct((nb, 1, block_e), jnp.float32),
    )(a_ij, W1, b1.reshape(1, -1), W2, b2.reshape(1, -1), Wh)
    return out.reshape(e)


_LANES = 16
_MAGIC = 0x5F3759DF


def _rsqrt16(x):
    i = plsc.bitcast(x, jnp.int32)
    i = _MAGIC - lax.shift_right_logical(i, 1)
    y = plsc.bitcast(i, jnp.float32)
    hx = 0.5 * x
    y = y * (1.5 - hx * y * y)
    y = y * (1.5 - hx * y * y)
    y = y * (1.5 - hx * y * y)
    return y


def _make_sc_edge(n, e, n_workers):
    ew = e // n_workers
    cw = 3 * n
    mesh = plsc.VectorSubcoreMesh(core_axis_name="c", subcore_axis_name="s")

    @functools.partial(
        pl.kernel,
        mesh=mesh,
        compiler_params=pltpu.CompilerParams(needs_layout_passes=False),
        out_type=jax.ShapeDtypeStruct((n_workers, cw), jnp.float32),
        scratch_types=[
            pltpu.VMEM((cw,), jnp.float32),
            pltpu.VMEM((cw,), jnp.float32),
            pltpu.VMEM((ew,), jnp.int32),
            pltpu.VMEM((ew,), jnp.int32),
            pltpu.VMEM((ew,), jnp.float32),
        ],
    )
    def sc_edge(coords_hbm, i_hbm, j_hbm, att_hbm, out_hbm,
                coords_v, acc_v, i_v, j_v, att_v):
        cid = lax.axis_index("c")
        sid = lax.axis_index("s")
        wid = sid * 2 + cid
        base = pl.multiple_of(wid * ew, 8)

        pltpu.sync_copy(coords_hbm, coords_v)
        pltpu.sync_copy(i_hbm.at[pl.ds(base, ew)], i_v)
        pltpu.sync_copy(j_hbm.at[pl.ds(base, ew)], j_v)
        pltpu.sync_copy(att_hbm.at[pl.ds(base, ew)], att_v)

        zeros = jnp.zeros((_LANES,), jnp.float32)

        @plsc.parallel_loop(0, cw, _LANES, unroll=8)
        def _(off):
            acc_v[pl.ds(off, _LANES)] = zeros

        @plsc.parallel_loop(0, ew, _LANES, unroll=4)
        def _(off):
            iv = i_v[pl.ds(off, _LANES)]
            jv = j_v[pl.ds(off, _LANES)]
            av = att_v[pl.ds(off, _LANES)]
            bi = iv * 3
            bj = jv * 3
            xi = plsc.load_gather(coords_v, [bi])
            yi = plsc.load_gather(coords_v, [bi + 1])
            zi = plsc.load_gather(coords_v, [bi + 2])
            xj = plsc.load_gather(coords_v, [bj])
            yj = plsc.load_gather(coords_v, [bj + 1])
            zj = plsc.load_gather(coords_v, [bj + 2])
            dx = xi - xj
            dy = yi - yj
            dz = zi - zj
            s2 = dx * dx + dy * dy + dz * dz
            s2 = jnp.maximum(s2, 1e-30)
            norm = s2 * _rsqrt16(s2)
            f = av / (norm + 1e-6)
            plsc.addupdate_scatter(acc_v, [bi], dx * f)
            plsc.addupdate_scatter(acc_v, [bi + 1], dy * f)
            plsc.addupdate_scatter(acc_v, [bi + 2], dz * f)

        pltpu.sync_copy(acc_v, out_hbm.at[wid])

    return sc_edge


def _reduce_body(p_ref, c_ref, o_ref):
    o_ref[...] = c_ref[...] + jnp.sum(p_ref[...], axis=0, keepdims=True)


def _reduce_partials(partials, coords_flat):
    nw, cw = partials.shape
    out = pl.pallas_call(
        _reduce_body,
        in_specs=[
            pl.BlockSpec((nw, cw), lambda: (0, 0)),
            pl.BlockSpec((1, cw), lambda: (0, 0)),
        ],
        out_specs=pl.BlockSpec((1, cw), lambda: (0, 0)),
        out_shape=jax.ShapeDtypeStruct((1, cw), jnp.float32),
    )(partials, coords_flat.reshape(1, cw))
    return out.reshape(cw)


def kernel(a_ij, coords, edge_index, W1, b1, W2, b2, Wh):
    e = a_ij.shape[0]
    n = coords.shape[0]
    att = _compute_att(a_ij, W1, b1, W2, b2, Wh, block_e=2560)
    coords_flat = coords.reshape(-1)
    sc_edge = _make_sc_edge(n, e, 32)
    partials = sc_edge(coords_flat, edge_index[0], edge_index[1], att)
    out_flat = _reduce_partials(partials, coords_flat)
    return out_flat.reshape(n, 3)
